# baseline (speedup 1.0000x reference)
import functools

import jax
import jax.numpy as jnp
from jax import lax
from jax.experimental import pallas as pl
from jax.experimental.pallas import tpu as pltpu
from jax.experimental.pallas import tpu_sc as plsc

B = 4
S = 2048
D = 768
H = 768
E = 64
K = 2
N = B * S
NK = N * K

TB = 512
NA = N // TB


def _router_body(x_ref, wg_ref, xg0_ref, xg1_ref, ii_ref):
    x = x_ref[:]
    logits = jnp.dot(x, wg_ref[:], preferred_element_type=jnp.float32)
    m = jnp.max(logits, axis=1, keepdims=True)
    ex = jnp.exp(logits - m)
    probs = ex / jnp.sum(ex, axis=1, keepdims=True)
    lane = lax.broadcasted_iota(jnp.int32, (TB, E), 1)
    p0 = jnp.max(probs, axis=1, keepdims=True)
    i0 = jnp.min(jnp.where(probs == p0, lane, E), axis=1, keepdims=True)
    probs2 = jnp.where(lane == i0, -jnp.inf, probs)
    p1 = jnp.max(probs2, axis=1, keepdims=True)
    i1 = jnp.min(jnp.where(probs2 == p1, lane, E), axis=1, keepdims=True)
    xg0_ref[:] = x * p0
    xg1_ref[:] = x * p1
    ii_ref[:] = jnp.concatenate([i0, i1], axis=1)


def _router(xf, Wg):
    return pl.pallas_call(
        _router_body,
        grid=(NA,),
        in_specs=[
            pl.BlockSpec((TB, D), lambda i: (i, 0)),
            pl.BlockSpec((D, E), lambda i: (0, 0)),
        ],
        out_specs=[
            pl.BlockSpec((TB, D), lambda i: (i, 0)),
            pl.BlockSpec((TB, D), lambda i: (i, 0)),
            pl.BlockSpec((TB, K), lambda i: (i, 0)),
        ],
        out_shape=[
            jax.ShapeDtypeStruct((N, D), jnp.float32),
            jax.ShapeDtypeStruct((N, D), jnp.float32),
            jax.ShapeDtypeStruct((N, K), jnp.int32),
        ],
    )(xf, Wg)


GC = 256
GR = NK // GC


def _sortpos_body(ids_ref, pos_ref, off_ref):
    ids2d = ids_ref[:]
    c0 = lax.broadcasted_iota(jnp.int32, (GC, GC), 0)
    c1 = lax.broadcasted_iota(jnp.int32, (GC, GC), 1)
    su = (c0 < c1).astype(jnp.float32)
    ones = jnp.ones((GC, GC), jnp.float32)
    g0 = lax.broadcasted_iota(jnp.int32, (GR, GR), 0)
    g1 = lax.broadcasted_iota(jnp.int32, (GR, GR), 1)
    slg = (g1 < g0).astype(jnp.float32)
    rowid = lax.broadcasted_iota(jnp.int32, (GR, GC), 0)

    def step(e, carry):
        pos_acc, off_mat, off_sc = carry
        mask = (ids2d == e).astype(jnp.float32)
        rowcnt = jnp.dot(mask, ones,
                         preferred_element_type=jnp.float32)
        pre = jnp.dot(slg, rowcnt,
                      preferred_element_type=jnp.float32)
        rank = jnp.dot(mask, su,
                       preferred_element_type=jnp.float32)
        pos_acc = pos_acc + mask * (off_sc + pre + rank)
        off_mat = off_mat + (rowid == e).astype(jnp.float32) * off_sc
        off_sc = off_sc + jnp.sum(mask)
        return pos_acc, off_mat, off_sc

    z = jnp.zeros((GR, GC), jnp.float32)
    pos_acc, off_mat, _ = lax.fori_loop(0, E, step, (z, z, 0.0))
    pos_ref[:] = pos_acc.astype(jnp.int32)
    off_ref[:] = off_mat.astype(jnp.int32)


def _sortpos(ids2):
    return pl.pallas_call(
        _sortpos_body,
        out_shape=[
            jax.ShapeDtypeStruct((GR, GC), jnp.int32),
            jax.ShapeDtypeStruct((GR, GC), jnp.int32),
        ],
    )(ids2)


TM = 256
NT = NK // TM
G = NT + E - 1


def _ffn_body(tid_ref, eid_ref, first_ref, valid_ref, off_ref,
              x_ref, w1_ref, w2_ref, out_ref):
    g = pl.program_id(0)

    @pl.when(valid_ref[g] == 1)
    def _():
        e = eid_ref[g]
        t = tid_ref[g]
        lo = off_ref[e]
        hi = off_ref[e + 1]
        rows = t * TM + lax.broadcasted_iota(jnp.int32, (TM, 1), 0)
        msk = (rows >= lo) & (rows < hi)
        x = jnp.where(msk, x_ref[:], 0.0).astype(jnp.bfloat16)
        h = jnp.maximum(
            jnp.dot(x, w1_ref[0], preferred_element_type=jnp.float32), 0.0)
        part = jnp.dot(h.astype(jnp.bfloat16), w2_ref[0],
                       preferred_element_type=jnp.float32)

        @pl.when(first_ref[g] == 1)
        def _():
            out_ref[:] = part

        @pl.when(first_ref[g] == 0)
        def _():
            out_ref[:] = out_ref[:] + part


def _ffn(tid, eid, first, valid, offsets, Xs, W1, W2):
    grid_spec = pltpu.PrefetchScalarGridSpec(
        num_scalar_prefetch=5,
        grid=(G,),
        in_specs=[
            pl.BlockSpec((TM, D), lambda g, t, e, f, v, o: (t[g], 0)),
            pl.BlockSpec((1, D, H), lambda g, t, e, f, v, o: (e[g], 0, 0)),
            pl.BlockSpec((1, H, D), lambda g, t, e, f, v, o: (e[g], 0, 0)),
        ],
        out_specs=pl.BlockSpec((TM, D), lambda g, t, e, f, v, o: (t[g], 0)),
    )
    return pl.pallas_call(
        _ffn_body,
        grid_spec=grid_spec,
        out_shape=jax.ShapeDtypeStruct((NK, D), jnp.float32),
    )(tid, eid, first, valid, offsets, Xs, W1, W2)


_NC, _NS = 2, 16
_NW = _NC * _NS
CH = 64
NCH = (N // _NW) // CH


def _sc_mesh():
    return plsc.VectorSubcoreMesh(core_axis_name="c", subcore_axis_name="s",
                                  num_cores=_NC, num_subcores=_NS)


def _scatter(xg0, xg1, pe3, po3):
    @functools.partial(
        pl.kernel,
        out_type=jax.ShapeDtypeStruct((NK, D), jnp.float32),
        mesh=_sc_mesh(),
        scratch_types=[
            pltpu.VMEM((CH, D), jnp.float32),
            pltpu.VMEM((NCH, CH), jnp.int32),
            pltpu.VMEM((NCH, CH), jnp.int32),
            pltpu.SemaphoreType.DMA,
        ],
    )
    def k(xg0_hbm, xg1_hbm, pe_hbm, po_hbm, out_hbm, rowbuf, idxe, idxo, sem):
        w = lax.axis_index("s") * _NC + lax.axis_index("c")
        base = w * (N // _NW)
        pltpu.sync_copy(pe_hbm.at[w], idxe)
        pltpu.sync_copy(po_hbm.at[w], idxo)
        for j in range(NCH):
            pltpu.sync_copy(xg0_hbm.at[pl.ds(base + j * CH, CH)], rowbuf)
            pltpu.async_copy(rowbuf, out_hbm.at[idxe.at[j]], sem).wait()
            pltpu.sync_copy(xg1_hbm.at[pl.ds(base + j * CH, CH)], rowbuf)
            pltpu.async_copy(rowbuf, out_hbm.at[idxo.at[j]], sem).wait()

    return k(xg0, xg1, pe3, po3)


def _combine(xout, pe3, po3):
    @functools.partial(
        pl.kernel,
        out_type=jax.ShapeDtypeStruct((N, D), jnp.float32),
        mesh=_sc_mesh(),
        scratch_types=[
            pltpu.VMEM((CH, D), jnp.float32),
            pltpu.VMEM((CH, D), jnp.float32),
            pltpu.VMEM((NCH, CH), jnp.int32),
            pltpu.VMEM((NCH, CH), jnp.int32),
            pltpu.SemaphoreType.DMA,
        ],
    )
    def k(xout_hbm, pe_hbm, po_hbm, y_hbm, buf0, buf1, idxe, idxo, sem):
        w = lax.axis_index("s") * _NC + lax.axis_index("c")
        base = w * (N // _NW)
        pltpu.sync_copy(pe_hbm.at[w], idxe)
        pltpu.sync_copy(po_hbm.at[w], idxo)
        for j in range(NCH):
            pltpu.async_copy(xout_hbm.at[idxe.at[j]], buf0, sem).wait()
            pltpu.async_copy(xout_hbm.at[idxo.at[j]], buf1, sem).wait()

            def rowadd(r, carry):
                for c in range(D // 16):
                    sl = pl.ds(c * 16, 16)
                    buf0[r, sl] = buf0[r, sl] + buf1[r, sl]
                return carry

            lax.fori_loop(0, CH, rowadd, 0)
            pltpu.sync_copy(buf0, y_hbm.at[pl.ds(base + j * CH, CH)])

    return k(xout, pe3, po3)


def kernel(inputBatch, Wg, W1, W2):
    xf = inputBatch.reshape(-1, D)
    xg0, xg1, ii = _router(xf, Wg)

    ids2 = ii.reshape(GR, GC)
    pos2, off2 = _sortpos(ids2)
    pos = pos2.reshape(-1)

    offsets = jnp.concatenate(
        [off2[:, 0], jnp.full((1,), NK, jnp.int32)])

    tt = jnp.arange(NT, dtype=jnp.int32)
    es = jnp.searchsorted(offsets, tt * TM, side="right").astype(jnp.int32) - 1
    ee = jnp.searchsorted(offsets, tt * TM + (TM - 1),
                          side="right").astype(jnp.int32) - 1
    ne = ee - es + 1
    start_g = jnp.concatenate(
        [jnp.zeros((1,), jnp.int32), jnp.cumsum(ne, dtype=jnp.int32)])
    total = start_g[-1]
    gg = jnp.arange(G, dtype=jnp.int32)
    tg = jnp.clip(
        jnp.searchsorted(start_g, gg, side="right").astype(jnp.int32) - 1,
        0, NT - 1)
    eg = es[tg] + (gg - start_g[tg])
    valid = gg < total
    first = jnp.where(valid, (gg == start_g[tg]), False).astype(jnp.int32)
    tg = jnp.where(valid, tg, NT - 1)
    eg = jnp.where(valid, jnp.clip(eg, 0, E - 1), E - 1)
    valid = valid.astype(jnp.int32)

    pe3 = pos[0::2].reshape(_NW, NCH, CH)
    po3 = pos[1::2].reshape(_NW, NCH, CH)

    Xs = _scatter(xg0, xg1, pe3, po3)
    Xout = _ffn(tg, eg, first, valid, offsets,
                Xs, W1.astype(jnp.bfloat16), W2.astype(jnp.bfloat16))
    y = _combine(Xout, pe3, po3)
    return y.reshape(B, S, D)


# speedup vs baseline: 1.2315x; 1.2315x over previous
import functools

import jax
import jax.numpy as jnp
from jax import lax
from jax.experimental import pallas as pl
from jax.experimental.pallas import tpu as pltpu
from jax.experimental.pallas import tpu_sc as plsc

B = 4
S = 2048
D = 768
H = 768
E = 64
K = 2
N = B * S
NK = N * K

TB = 512
NA = N // TB


def _router_body(x_ref, wg_ref, xg0_ref, xg1_ref, ii_ref):
    x = x_ref[:]
    logits = jnp.dot(x, wg_ref[:], preferred_element_type=jnp.float32)
    m = jnp.max(logits, axis=1, keepdims=True)
    ex = jnp.exp(logits - m)
    probs = ex / jnp.sum(ex, axis=1, keepdims=True)
    lane = lax.broadcasted_iota(jnp.int32, (TB, E), 1)
    p0 = jnp.max(probs, axis=1, keepdims=True)
    i0 = jnp.min(jnp.where(probs == p0, lane, E), axis=1, keepdims=True)
    probs2 = jnp.where(lane == i0, -jnp.inf, probs)
    p1 = jnp.max(probs2, axis=1, keepdims=True)
    i1 = jnp.min(jnp.where(probs2 == p1, lane, E), axis=1, keepdims=True)
    xg0_ref[:] = x * p0
    xg1_ref[:] = x * p1
    ii_ref[:] = jnp.concatenate([i0, i1], axis=1)


def _router(xf, Wg):
    return pl.pallas_call(
        _router_body,
        grid=(NA,),
        in_specs=[
            pl.BlockSpec((TB, D), lambda i: (i, 0)),
            pl.BlockSpec((D, E), lambda i: (0, 0)),
        ],
        out_specs=[
            pl.BlockSpec((TB, D), lambda i: (i, 0)),
            pl.BlockSpec((TB, D), lambda i: (i, 0)),
            pl.BlockSpec((TB, K), lambda i: (i, 0)),
        ],
        out_shape=[
            jax.ShapeDtypeStruct((N, D), jnp.float32),
            jax.ShapeDtypeStruct((N, D), jnp.float32),
            jax.ShapeDtypeStruct((N, K), jnp.int32),
        ],
    )(xf, Wg)


GC = 256
GR = NK // GC


def _sortpos_body(ids_ref, pos_ref, off_ref):
    ids2d = ids_ref[:]
    c0 = lax.broadcasted_iota(jnp.int32, (GC, GC), 0)
    c1 = lax.broadcasted_iota(jnp.int32, (GC, GC), 1)
    su = (c0 < c1).astype(jnp.bfloat16)
    ones = jnp.ones((GC, GC), jnp.bfloat16)
    g0 = lax.broadcasted_iota(jnp.int32, (GR, GR), 0)
    g1 = lax.broadcasted_iota(jnp.int32, (GR, GR), 1)
    slg = (g1 < g0).astype(jnp.bfloat16)
    rowid = lax.broadcasted_iota(jnp.int32, (GR, GC), 0)

    def step(e, carry):
        pos_acc, off_mat, off_sc = carry
        maskb = (ids2d == e).astype(jnp.bfloat16)
        mask = maskb.astype(jnp.float32)
        rowcnt = jnp.dot(maskb, ones,
                         preferred_element_type=jnp.float32)
        pre = jnp.dot(slg, rowcnt.astype(jnp.bfloat16),
                      preferred_element_type=jnp.float32)
        rank = jnp.dot(maskb, su,
                       preferred_element_type=jnp.float32)
        pos_acc = pos_acc + mask * (off_sc + pre + rank)
        off_mat = off_mat + (rowid == e).astype(jnp.float32) * off_sc
        off_sc = off_sc + jnp.sum(mask)
        return pos_acc, off_mat, off_sc

    z = jnp.zeros((GR, GC), jnp.float32)
    pos_acc, off_mat, _ = lax.fori_loop(0, E, step, (z, z, 0.0))
    pos_ref[:] = pos_acc.astype(jnp.int32)
    off_ref[:] = off_mat.astype(jnp.int32)


def _sortpos(ids2):
    return pl.pallas_call(
        _sortpos_body,
        out_shape=[
            jax.ShapeDtypeStruct((GR, GC), jnp.int32),
            jax.ShapeDtypeStruct((GR, GC), jnp.int32),
        ],
    )(ids2)


TM = 256
NT = NK // TM
G = NT + E - 1


def _ffn_body(tid_ref, eid_ref, first_ref, valid_ref, off_ref,
              x_ref, w1_ref, w2_ref, out_ref):
    g = pl.program_id(0)

    @pl.when(valid_ref[g] == 1)
    def _():
        e = eid_ref[g]
        t = tid_ref[g]
        lo = off_ref[e]
        hi = off_ref[e + 1]
        rows = t * TM + lax.broadcasted_iota(jnp.int32, (TM, 1), 0)
        msk = (rows >= lo) & (rows < hi)
        x = jnp.where(msk, x_ref[:], 0.0)
        h = jnp.maximum(
            jnp.dot(x, w1_ref[0], preferred_element_type=jnp.float32), 0.0)
        part = jnp.dot(h, w2_ref[0], preferred_element_type=jnp.float32)

        @pl.when(first_ref[g] == 1)
        def _():
            out_ref[:] = part

        @pl.when(first_ref[g] == 0)
        def _():
            out_ref[:] = out_ref[:] + part


def _ffn(tid, eid, first, valid, offsets, Xs, W1, W2):
    grid_spec = pltpu.PrefetchScalarGridSpec(
        num_scalar_prefetch=5,
        grid=(G,),
        in_specs=[
            pl.BlockSpec((TM, D), lambda g, t, e, f, v, o: (t[g], 0)),
            pl.BlockSpec((1, D, H), lambda g, t, e, f, v, o: (e[g], 0, 0)),
            pl.BlockSpec((1, H, D), lambda g, t, e, f, v, o: (e[g], 0, 0)),
        ],
        out_specs=pl.BlockSpec((TM, D), lambda g, t, e, f, v, o: (t[g], 0)),
    )
    return pl.pallas_call(
        _ffn_body,
        grid_spec=grid_spec,
        out_shape=jax.ShapeDtypeStruct((NK, D), jnp.float32),
    )(tid, eid, first, valid, offsets, Xs, W1, W2)


_NC, _NS = 2, 16
_NW = _NC * _NS
CH = 64
NCH = (N // _NW) // CH


def _sc_mesh():
    return plsc.VectorSubcoreMesh(core_axis_name="c", subcore_axis_name="s",
                                  num_cores=_NC, num_subcores=_NS)


def _scatter(xg0, xg1, pe3, po3):
    @functools.partial(
        pl.kernel,
        out_type=jax.ShapeDtypeStruct((NK, D), jnp.float32),
        mesh=_sc_mesh(),
        scratch_types=[
            pltpu.VMEM((CH, D), jnp.float32),
            pltpu.VMEM((NCH, CH), jnp.int32),
            pltpu.VMEM((NCH, CH), jnp.int32),
            pltpu.SemaphoreType.DMA,
        ],
    )
    def k(xg0_hbm, xg1_hbm, pe_hbm, po_hbm, out_hbm, rowbuf, idxe, idxo, sem):
        w = lax.axis_index("s") * _NC + lax.axis_index("c")
        base = w * (N // _NW)
        pltpu.sync_copy(pe_hbm.at[w], idxe)
        pltpu.sync_copy(po_hbm.at[w], idxo)
        for j in range(NCH):
            pltpu.sync_copy(xg0_hbm.at[pl.ds(base + j * CH, CH)], rowbuf)
            pltpu.async_copy(rowbuf, out_hbm.at[idxe.at[j]], sem).wait()
            pltpu.sync_copy(xg1_hbm.at[pl.ds(base + j * CH, CH)], rowbuf)
            pltpu.async_copy(rowbuf, out_hbm.at[idxo.at[j]], sem).wait()

    return k(xg0, xg1, pe3, po3)


def _combine(xout, pe3, po3):
    @functools.partial(
        pl.kernel,
        out_type=jax.ShapeDtypeStruct((N, D), jnp.float32),
        mesh=_sc_mesh(),
        scratch_types=[
            pltpu.VMEM((CH, D), jnp.float32),
            pltpu.VMEM((CH, D), jnp.float32),
            pltpu.VMEM((NCH, CH), jnp.int32),
            pltpu.VMEM((NCH, CH), jnp.int32),
            pltpu.SemaphoreType.DMA,
        ],
    )
    def k(xout_hbm, pe_hbm, po_hbm, y_hbm, buf0, buf1, idxe, idxo, sem):
        w = lax.axis_index("s") * _NC + lax.axis_index("c")
        base = w * (N // _NW)
        pltpu.sync_copy(pe_hbm.at[w], idxe)
        pltpu.sync_copy(po_hbm.at[w], idxo)
        for j in range(NCH):
            pltpu.async_copy(xout_hbm.at[idxe.at[j]], buf0, sem).wait()
            pltpu.async_copy(xout_hbm.at[idxo.at[j]], buf1, sem).wait()

            def rowadd(r, carry):
                for c in range(D // 16):
                    sl = pl.ds(c * 16, 16)
                    buf0[r, sl] = buf0[r, sl] + buf1[r, sl]
                return carry

            lax.fori_loop(0, CH, rowadd, 0)
            pltpu.sync_copy(buf0, y_hbm.at[pl.ds(base + j * CH, CH)])

    return k(xout, pe3, po3)


def kernel(inputBatch, Wg, W1, W2):
    xf = inputBatch.reshape(-1, D)
    xg0, xg1, ii = _router(xf, Wg)

    ids2 = ii.reshape(GR, GC)
    pos2, off2 = _sortpos(ids2)
    pos = pos2.reshape(-1)

    offsets = jnp.concatenate(
        [off2[:, 0], jnp.full((1,), NK, jnp.int32)])

    tt = jnp.arange(NT, dtype=jnp.int32)
    es = jnp.searchsorted(offsets, tt * TM, side="right").astype(jnp.int32) - 1
    ee = jnp.searchsorted(offsets, tt * TM + (TM - 1),
                          side="right").astype(jnp.int32) - 1
    ne = ee - es + 1
    start_g = jnp.concatenate(
        [jnp.zeros((1,), jnp.int32), jnp.cumsum(ne, dtype=jnp.int32)])
    total = start_g[-1]
    gg = jnp.arange(G, dtype=jnp.int32)
    tg = jnp.clip(
        jnp.searchsorted(start_g, gg, side="right").astype(jnp.int32) - 1,
        0, NT - 1)
    eg = es[tg] + (gg - start_g[tg])
    valid = gg < total
    first = jnp.where(valid, (gg == start_g[tg]), False).astype(jnp.int32)
    tg = jnp.where(valid, tg, NT - 1)
    eg = jnp.where(valid, jnp.clip(eg, 0, E - 1), E - 1)
    valid = valid.astype(jnp.int32)

    pe3 = pos[0::2].reshape(_NW, NCH, CH)
    po3 = pos[1::2].reshape(_NW, NCH, CH)

    Xs = _scatter(xg0, xg1, pe3, po3)
    Xout = _ffn(tg, eg, first, valid, offsets, Xs, W1, W2)
    y = _combine(Xout, pe3, po3)
    return y.reshape(B, S, D)


# speedup vs baseline: 1.2867x; 1.0449x over previous
import functools

import jax
import jax.numpy as jnp
from jax import lax
from jax.experimental import pallas as pl
from jax.experimental.pallas import tpu as pltpu
from jax.experimental.pallas import tpu_sc as plsc

B = 4
S = 2048
D = 768
H = 768
E = 64
K = 2
N = B * S
NK = N * K

TB = 512
NA = N // TB


def _router_body(x_ref, wg_ref, xg0_ref, xg1_ref, ii_ref):
    x = x_ref[:]
    logits = jnp.dot(x, wg_ref[:], preferred_element_type=jnp.float32)
    m = jnp.max(logits, axis=1, keepdims=True)
    ex = jnp.exp(logits - m)
    probs = ex / jnp.sum(ex, axis=1, keepdims=True)
    lane = lax.broadcasted_iota(jnp.int32, (TB, E), 1)
    p0 = jnp.max(probs, axis=1, keepdims=True)
    i0 = jnp.min(jnp.where(probs == p0, lane, E), axis=1, keepdims=True)
    probs2 = jnp.where(lane == i0, -jnp.inf, probs)
    p1 = jnp.max(probs2, axis=1, keepdims=True)
    i1 = jnp.min(jnp.where(probs2 == p1, lane, E), axis=1, keepdims=True)
    xg0_ref[:] = x * p0
    xg1_ref[:] = x * p1
    ii_ref[:] = jnp.concatenate([i0, i1], axis=1)


def _router(xf, Wg):
    return pl.pallas_call(
        _router_body,
        grid=(NA,),
        in_specs=[
            pl.BlockSpec((TB, D), lambda i: (i, 0)),
            pl.BlockSpec((D, E), lambda i: (0, 0)),
        ],
        out_specs=[
            pl.BlockSpec((TB, D), lambda i: (i, 0)),
            pl.BlockSpec((TB, D), lambda i: (i, 0)),
            pl.BlockSpec((TB, K), lambda i: (i, 0)),
        ],
        out_shape=[
            jax.ShapeDtypeStruct((N, D), jnp.float32),
            jax.ShapeDtypeStruct((N, D), jnp.float32),
            jax.ShapeDtypeStruct((N, K), jnp.int32),
        ],
    )(xf, Wg)


GC = 256
GR = NK // GC


def _sortpos_body(ids_ref, pos_ref, off_ref):
    ids2d = ids_ref[:]
    c0 = lax.broadcasted_iota(jnp.int32, (GC, GC), 0)
    c1 = lax.broadcasted_iota(jnp.int32, (GC, GC), 1)
    su = (c0 < c1).astype(jnp.bfloat16)
    ones = jnp.ones((GC, GC), jnp.bfloat16)
    g0 = lax.broadcasted_iota(jnp.int32, (GR, GR), 0)
    g1 = lax.broadcasted_iota(jnp.int32, (GR, GR), 1)
    slg = (g1 < g0).astype(jnp.bfloat16)
    rowid = lax.broadcasted_iota(jnp.int32, (GR, GC), 0)

    def step(e, carry):
        pos_acc, off_mat, off_sc = carry
        maskb = (ids2d == e).astype(jnp.bfloat16)
        mask = maskb.astype(jnp.float32)
        rowcnt = jnp.dot(maskb, ones,
                         preferred_element_type=jnp.float32)
        pre = jnp.dot(slg, rowcnt.astype(jnp.bfloat16),
                      preferred_element_type=jnp.float32)
        rank = jnp.dot(maskb, su,
                       preferred_element_type=jnp.float32)
        pos_acc = pos_acc + mask * (off_sc + pre + rank)
        off_mat = off_mat + (rowid == e).astype(jnp.float32) * off_sc
        off_sc = off_sc + jnp.sum(mask)
        return pos_acc, off_mat, off_sc

    z = jnp.zeros((GR, GC), jnp.float32)
    pos_acc, off_mat, _ = lax.fori_loop(0, E, step, (z, z, 0.0))
    pos_ref[:] = pos_acc.astype(jnp.int32)
    off_ref[:] = off_mat.astype(jnp.int32)


def _sortpos(ids2):
    return pl.pallas_call(
        _sortpos_body,
        out_shape=[
            jax.ShapeDtypeStruct((GR, GC), jnp.int32),
            jax.ShapeDtypeStruct((GR, GC), jnp.int32),
        ],
    )(ids2)


TM = 256
NT = NK // TM
G = NT + E - 1


def _ffn_body(tid_ref, eid_ref, first_ref, valid_ref, off_ref,
              x_ref, w1_ref, w2_ref, out_ref):
    g = pl.program_id(0)

    @pl.when(valid_ref[g] == 1)
    def _():
        e = eid_ref[g]
        t = tid_ref[g]
        lo = off_ref[e]
        hi = off_ref[e + 1]
        rows = t * TM + lax.broadcasted_iota(jnp.int32, (TM, 1), 0)
        msk = (rows >= lo) & (rows < hi)
        x = jnp.where(msk, x_ref[:], 0.0)
        h = jnp.maximum(
            jnp.dot(x, w1_ref[0], preferred_element_type=jnp.float32), 0.0)
        part = jnp.dot(h, w2_ref[0], preferred_element_type=jnp.float32)

        @pl.when(first_ref[g] == 1)
        def _():
            out_ref[:] = part

        @pl.when(first_ref[g] == 0)
        def _():
            out_ref[:] = out_ref[:] + part


def _ffn(tid, eid, first, valid, offsets, Xs, W1, W2):
    grid_spec = pltpu.PrefetchScalarGridSpec(
        num_scalar_prefetch=5,
        grid=(G,),
        in_specs=[
            pl.BlockSpec((TM, D), lambda g, t, e, f, v, o: (t[g], 0)),
            pl.BlockSpec((1, D, H), lambda g, t, e, f, v, o: (e[g], 0, 0)),
            pl.BlockSpec((1, H, D), lambda g, t, e, f, v, o: (e[g], 0, 0)),
        ],
        out_specs=pl.BlockSpec((TM, D), lambda g, t, e, f, v, o: (t[g], 0)),
    )
    return pl.pallas_call(
        _ffn_body,
        grid_spec=grid_spec,
        out_shape=jax.ShapeDtypeStruct((NK, D), jnp.float32),
    )(tid, eid, first, valid, offsets, Xs, W1, W2)


_NC, _NS = 2, 16
_NW = _NC * _NS
CH = 64
NCH = (N // _NW) // CH
CHE = 32
NCHE = (N // _NW) // CHE


def _sc_mesh():
    return plsc.VectorSubcoreMesh(core_axis_name="c", subcore_axis_name="s",
                                  num_cores=_NC, num_subcores=_NS)


def _scatter(xg0, xg1, pe3, po3):
    @functools.partial(
        pl.kernel,
        out_type=jax.ShapeDtypeStruct((NK, D), jnp.float32),
        mesh=_sc_mesh(),
        scratch_types=[
            pltpu.VMEM((CH, D), jnp.float32),
            pltpu.VMEM((CH, D), jnp.float32),
            pltpu.VMEM((NCH, CH), jnp.int32),
            pltpu.VMEM((NCH, CH), jnp.int32),
            pltpu.SemaphoreType.DMA,
            pltpu.SemaphoreType.DMA,
            pltpu.SemaphoreType.DMA,
            pltpu.SemaphoreType.DMA,
        ],
    )
    def k(xg0_hbm, xg1_hbm, pe_hbm, po_hbm, out_hbm,
          rb0, rb1, idxe, idxo, ls0, ls1, ss0, ss1):
        w = lax.axis_index("s") * _NC + lax.axis_index("c")
        base = w * (N // _NW)
        pltpu.sync_copy(pe_hbm.at[w], idxe)
        pltpu.sync_copy(po_hbm.at[w], idxo)
        tasks = ([(xg0_hbm, idxe, j) for j in range(NCH)]
                 + [(xg1_hbm, idxo, j) for j in range(NCH)])
        bufs, lsems, ssems = (rb0, rb1), (ls0, ls1), (ss0, ss1)
        nt = len(tasks)

        def start_load(i):
            srcref, _, j = tasks[i]
            return pltpu.async_copy(
                srcref.at[pl.ds(base + j * CH, CH)], bufs[i % 2],
                lsems[i % 2])

        loads = {0: start_load(0)}
        scats = {}
        for i in range(nt):
            if i + 1 < nt:
                if i - 1 >= 0:
                    scats[i - 1].wait()
                loads[i + 1] = start_load(i + 1)
            loads[i].wait()
            _, idxref, j = tasks[i]
            scats[i] = pltpu.async_copy(
                bufs[i % 2], out_hbm.at[idxref.at[j]], ssems[i % 2])
        scats[nt - 2].wait()
        scats[nt - 1].wait()

    return k(xg0, xg1, pe3, po3)


def _combine(xout, pe3, po3):
    @functools.partial(
        pl.kernel,
        out_type=jax.ShapeDtypeStruct((N, D), jnp.float32),
        mesh=_sc_mesh(),
        scratch_types=[
            pltpu.VMEM((CHE, D), jnp.float32),
            pltpu.VMEM((CHE, D), jnp.float32),
            pltpu.VMEM((CHE, D), jnp.float32),
            pltpu.VMEM((CHE, D), jnp.float32),
            pltpu.VMEM((NCHE, CHE), jnp.int32),
            pltpu.VMEM((NCHE, CHE), jnp.int32),
            pltpu.SemaphoreType.DMA,
            pltpu.SemaphoreType.DMA,
            pltpu.SemaphoreType.DMA,
            pltpu.SemaphoreType.DMA,
            pltpu.SemaphoreType.DMA,
            pltpu.SemaphoreType.DMA,
        ],
    )
    def k(xout_hbm, pe_hbm, po_hbm, y_hbm,
          ge0, ge1, go0, go1, idxe, idxo,
          gse0, gse1, gso0, gso1, sts0, sts1):
        w = lax.axis_index("s") * _NC + lax.axis_index("c")
        base = w * (N // _NW)
        pltpu.sync_copy(pe_hbm.at[w], idxe)
        pltpu.sync_copy(po_hbm.at[w], idxo)
        ge, go = (ge0, ge1), (go0, go1)
        gse, gso, sts = (gse0, gse1), (gso0, gso1), (sts0, sts1)

        def start_gather(j):
            p = j % 2
            return (pltpu.async_copy(xout_hbm.at[idxe.at[j]], ge[p], gse[p]),
                    pltpu.async_copy(xout_hbm.at[idxo.at[j]], go[p], gso[p]))

        gaths = {0: start_gather(0)}
        stores = {}
        for j in range(NCHE):
            p = j % 2
            if j + 1 < NCHE:
                if j - 1 >= 0:
                    stores[j - 1].wait()
                gaths[j + 1] = start_gather(j + 1)
            gaths[j][0].wait()
            gaths[j][1].wait()

            def rowadd(r, carry):
                for c in range(D // 16):
                    sl = pl.ds(c * 16, 16)
                    ge[p][r, sl] = ge[p][r, sl] + go[p][r, sl]
                return carry

            lax.fori_loop(0, CHE, rowadd, 0)
            stores[j] = pltpu.async_copy(
                ge[p], y_hbm.at[pl.ds(base + j * CHE, CHE)], sts[p])
        stores[NCHE - 2].wait()
        stores[NCHE - 1].wait()

    return k(xout, pe3, po3)


def kernel(inputBatch, Wg, W1, W2):
    xf = inputBatch.reshape(-1, D)
    xg0, xg1, ii = _router(xf, Wg)

    ids2 = ii.reshape(GR, GC)
    pos2, off2 = _sortpos(ids2)
    pos = pos2.reshape(-1)

    offsets = jnp.concatenate(
        [off2[:, 0], jnp.full((1,), NK, jnp.int32)])

    tt = jnp.arange(NT, dtype=jnp.int32)
    es = jnp.searchsorted(offsets, tt * TM, side="right").astype(jnp.int32) - 1
    ee = jnp.searchsorted(offsets, tt * TM + (TM - 1),
                          side="right").astype(jnp.int32) - 1
    ne = ee - es + 1
    start_g = jnp.concatenate(
        [jnp.zeros((1,), jnp.int32), jnp.cumsum(ne, dtype=jnp.int32)])
    total = start_g[-1]
    gg = jnp.arange(G, dtype=jnp.int32)
    tg = jnp.clip(
        jnp.searchsorted(start_g, gg, side="right").astype(jnp.int32) - 1,
        0, NT - 1)
    eg = es[tg] + (gg - start_g[tg])
    valid = gg < total
    first = jnp.where(valid, (gg == start_g[tg]), False).astype(jnp.int32)
    tg = jnp.where(valid, tg, NT - 1)
    eg = jnp.where(valid, jnp.clip(eg, 0, E - 1), E - 1)
    valid = valid.astype(jnp.int32)

    pos_e = pos[0::2]
    pos_o = pos[1::2]
    pe3 = pos_e.reshape(_NW, NCH, CH)
    po3 = pos_o.reshape(_NW, NCH, CH)
    pe3e = pos_e.reshape(_NW, NCHE, CHE)
    po3e = pos_o.reshape(_NW, NCHE, CHE)

    Xs = _scatter(xg0, xg1, pe3, po3)
    Xout = _ffn(tg, eg, first, valid, offsets, Xs, W1, W2)
    y = _combine(Xout, pe3e, po3e)
    return y.reshape(B, S, D)


# speedup vs baseline: 1.2953x; 1.0067x over previous
import functools

import jax
import jax.numpy as jnp
from jax import lax
from jax.experimental import pallas as pl
from jax.experimental.pallas import tpu as pltpu
from jax.experimental.pallas import tpu_sc as plsc

B = 4
S = 2048
D = 768
H = 768
E = 64
K = 2
N = B * S
NK = N * K

TB = 512
NA = N // TB


def _router_body(x_ref, wg_ref, xg0_ref, xg1_ref, ii_ref):
    x = x_ref[:]
    logits = jnp.dot(x, wg_ref[:], preferred_element_type=jnp.float32)
    m = jnp.max(logits, axis=1, keepdims=True)
    ex = jnp.exp(logits - m)
    probs = ex / jnp.sum(ex, axis=1, keepdims=True)
    lane = lax.broadcasted_iota(jnp.int32, (TB, E), 1)
    p0 = jnp.max(probs, axis=1, keepdims=True)
    i0 = jnp.min(jnp.where(probs == p0, lane, E), axis=1, keepdims=True)
    probs2 = jnp.where(lane == i0, -jnp.inf, probs)
    p1 = jnp.max(probs2, axis=1, keepdims=True)
    i1 = jnp.min(jnp.where(probs2 == p1, lane, E), axis=1, keepdims=True)
    xg0_ref[:] = x * p0
    xg1_ref[:] = x * p1
    ii_ref[:] = jnp.concatenate([i0, i1], axis=1)


def _router(xf, Wg):
    return pl.pallas_call(
        _router_body,
        grid=(NA,),
        in_specs=[
            pl.BlockSpec((TB, D), lambda i: (i, 0)),
            pl.BlockSpec((D, E), lambda i: (0, 0)),
        ],
        out_specs=[
            pl.BlockSpec((TB, D), lambda i: (i, 0)),
            pl.BlockSpec((TB, D), lambda i: (i, 0)),
            pl.BlockSpec((TB, K), lambda i: (i, 0)),
        ],
        out_shape=[
            jax.ShapeDtypeStruct((N, D), jnp.float32),
            jax.ShapeDtypeStruct((N, D), jnp.float32),
            jax.ShapeDtypeStruct((N, K), jnp.int32),
        ],
    )(xf, Wg)


GC = 256
GR = NK // GC


def _sortpos_body(ids_ref, pos_ref, off_ref):
    ids2d = ids_ref[:]
    c0 = lax.broadcasted_iota(jnp.int32, (GC, GC), 0)
    c1 = lax.broadcasted_iota(jnp.int32, (GC, GC), 1)
    su = (c0 < c1).astype(jnp.bfloat16)
    ones = jnp.ones((GC, GC), jnp.bfloat16)
    g0 = lax.broadcasted_iota(jnp.int32, (GR, GR), 0)
    g1 = lax.broadcasted_iota(jnp.int32, (GR, GR), 1)
    slg = (g1 < g0).astype(jnp.bfloat16)
    rowid = lax.broadcasted_iota(jnp.int32, (GR, GC), 0)

    def step(e, carry):
        pos_acc, off_mat, off_sc = carry
        maskb = (ids2d == e).astype(jnp.bfloat16)
        mask = maskb.astype(jnp.float32)
        rowcnt = jnp.dot(maskb, ones,
                         preferred_element_type=jnp.float32)
        pre = jnp.dot(slg, rowcnt.astype(jnp.bfloat16),
                      preferred_element_type=jnp.float32)
        rank = jnp.dot(maskb, su,
                       preferred_element_type=jnp.float32)
        pos_acc = pos_acc + mask * (off_sc + pre + rank)
        off_mat = off_mat + (rowid == e).astype(jnp.float32) * off_sc
        off_sc = off_sc + jnp.sum(mask)
        return pos_acc, off_mat, off_sc

    z = jnp.zeros((GR, GC), jnp.float32)
    pos_acc, off_mat, _ = lax.fori_loop(0, E, step, (z, z, 0.0))
    pos_ref[:] = pos_acc.astype(jnp.int32)
    off_ref[:] = off_mat.astype(jnp.int32)


def _sortpos(ids2):
    return pl.pallas_call(
        _sortpos_body,
        out_shape=[
            jax.ShapeDtypeStruct((GR, GC), jnp.int32),
            jax.ShapeDtypeStruct((GR, GC), jnp.int32),
        ],
    )(ids2)


TM = 256
NT = NK // TM
G = NT + E - 1


def _ffn_body(tid_ref, eid_ref, first_ref, valid_ref, off_ref,
              x_ref, w1_ref, w2_ref, out_ref):
    g = pl.program_id(0)

    @pl.when(valid_ref[g] == 1)
    def _():
        e = eid_ref[g]
        t = tid_ref[g]
        lo = off_ref[e]
        hi = off_ref[e + 1]
        rows = t * TM + lax.broadcasted_iota(jnp.int32, (TM, 1), 0)
        msk = (rows >= lo) & (rows < hi)
        x = jnp.where(msk, x_ref[:], 0.0)
        h = jnp.maximum(
            jnp.dot(x, w1_ref[0], preferred_element_type=jnp.float32), 0.0)
        part = jnp.dot(h, w2_ref[0], preferred_element_type=jnp.float32)

        @pl.when(first_ref[g] == 1)
        def _():
            out_ref[:] = part

        @pl.when(first_ref[g] == 0)
        def _():
            out_ref[:] = out_ref[:] + part


def _ffn(tid, eid, first, valid, offsets, Xs, W1, W2):
    grid_spec = pltpu.PrefetchScalarGridSpec(
        num_scalar_prefetch=5,
        grid=(G,),
        in_specs=[
            pl.BlockSpec((TM, D), lambda g, t, e, f, v, o: (t[g], 0)),
            pl.BlockSpec((1, D, H), lambda g, t, e, f, v, o: (e[g], 0, 0)),
            pl.BlockSpec((1, H, D), lambda g, t, e, f, v, o: (e[g], 0, 0)),
        ],
        out_specs=pl.BlockSpec((TM, D), lambda g, t, e, f, v, o: (t[g], 0)),
    )
    return pl.pallas_call(
        _ffn_body,
        grid_spec=grid_spec,
        out_shape=jax.ShapeDtypeStruct((NK, D), jnp.float32),
    )(tid, eid, first, valid, offsets, Xs, W1, W2)


_NC, _NS = 2, 16
_NW = _NC * _NS
CH = 64
NCH = (N // _NW) // CH
CHE = 32
NCHE = (N // _NW) // CHE


def _sc_mesh():
    return plsc.VectorSubcoreMesh(core_axis_name="c", subcore_axis_name="s",
                                  num_cores=_NC, num_subcores=_NS)


def _scatter(xg0, xg1, pe3, po3):
    @functools.partial(
        pl.kernel,
        out_type=jax.ShapeDtypeStruct((NK, D), jnp.float32),
        mesh=_sc_mesh(),
        scratch_types=[
            pltpu.VMEM((CH, D), jnp.float32),
            pltpu.VMEM((CH, D), jnp.float32),
            pltpu.VMEM((NCH, CH), jnp.int32),
            pltpu.VMEM((NCH, CH), jnp.int32),
            pltpu.SemaphoreType.DMA,
            pltpu.SemaphoreType.DMA,
            pltpu.SemaphoreType.DMA,
            pltpu.SemaphoreType.DMA,
        ],
    )
    def k(xg0_hbm, xg1_hbm, pe_hbm, po_hbm, out_hbm,
          rb0, rb1, idxe, idxo, ls0, ls1, ss0, ss1):
        w = lax.axis_index("s") * _NC + lax.axis_index("c")
        base = w * (N // _NW)
        pltpu.sync_copy(pe_hbm.at[w], idxe)
        pltpu.sync_copy(po_hbm.at[w], idxo)
        tasks = ([(xg0_hbm, idxe, j) for j in range(NCH)]
                 + [(xg1_hbm, idxo, j) for j in range(NCH)])
        bufs, lsems, ssems = (rb0, rb1), (ls0, ls1), (ss0, ss1)
        nt = len(tasks)

        def start_load(i):
            srcref, _, j = tasks[i]
            return pltpu.async_copy(
                srcref.at[pl.ds(base + j * CH, CH)], bufs[i % 2],
                lsems[i % 2])

        loads = {0: start_load(0)}
        scats = {}
        for i in range(nt):
            if i + 1 < nt:
                if i - 1 >= 0:
                    scats[i - 1].wait()
                loads[i + 1] = start_load(i + 1)
            loads[i].wait()
            _, idxref, j = tasks[i]
            scats[i] = pltpu.async_copy(
                bufs[i % 2], out_hbm.at[idxref.at[j]], ssems[i % 2])
        scats[nt - 2].wait()
        scats[nt - 1].wait()

    return k(xg0, xg1, pe3, po3)


def _combine(xout, pe3, po3):
    @functools.partial(
        pl.kernel,
        out_type=jax.ShapeDtypeStruct((N, D), jnp.float32),
        mesh=_sc_mesh(),
        scratch_types=[
            pltpu.VMEM((CHE, D), jnp.float32),
            pltpu.VMEM((CHE, D), jnp.float32),
            pltpu.VMEM((CHE, D), jnp.float32),
            pltpu.VMEM((CHE, D), jnp.float32),
            pltpu.VMEM((NCHE, CHE), jnp.int32),
            pltpu.VMEM((NCHE, CHE), jnp.int32),
            pltpu.SemaphoreType.DMA,
            pltpu.SemaphoreType.DMA,
            pltpu.SemaphoreType.DMA,
            pltpu.SemaphoreType.DMA,
            pltpu.SemaphoreType.DMA,
            pltpu.SemaphoreType.DMA,
        ],
    )
    def k(xout_hbm, pe_hbm, po_hbm, y_hbm,
          ge0, ge1, go0, go1, idxe, idxo,
          gse0, gse1, gso0, gso1, sts0, sts1):
        w = lax.axis_index("s") * _NC + lax.axis_index("c")
        base = w * (N // _NW)
        pltpu.sync_copy(pe_hbm.at[w], idxe)
        pltpu.sync_copy(po_hbm.at[w], idxo)
        ge, go = (ge0, ge1), (go0, go1)
        gse, gso, sts = (gse0, gse1), (gso0, gso1), (sts0, sts1)

        def start_gather(j):
            p = j % 2
            return (pltpu.async_copy(xout_hbm.at[idxe.at[j]], ge[p], gse[p]),
                    pltpu.async_copy(xout_hbm.at[idxo.at[j]], go[p], gso[p]))

        gaths = {0: start_gather(0)}
        stores = {}
        for j in range(NCHE):
            p = j % 2
            if j + 1 < NCHE:
                if j - 1 >= 0:
                    stores[j - 1].wait()
                gaths[j + 1] = start_gather(j + 1)
            gaths[j][0].wait()
            gaths[j][1].wait()

            def rowadd(r, carry):
                for c in range(D // 16):
                    sl = pl.ds(c * 16, 16)
                    ge[p][r, sl] = ge[p][r, sl] + go[p][r, sl]
                return carry

            lax.fori_loop(0, CHE, rowadd, 0)
            stores[j] = pltpu.async_copy(
                ge[p], y_hbm.at[pl.ds(base + j * CHE, CHE)], sts[p])
        stores[NCHE - 2].wait()
        stores[NCHE - 1].wait()

    return k(xout, pe3, po3)


def kernel(inputBatch, Wg, W1, W2):
    xf = inputBatch.reshape(-1, D)
    xg0, xg1, ii = _router(xf, Wg)

    ids2 = ii.reshape(GR, GC)
    pos2, off2 = _sortpos(ids2)
    pos = pos2.reshape(-1)

    offsets = jnp.concatenate(
        [off2[:, 0], jnp.full((1,), NK, jnp.int32)])

    tt = jnp.arange(NT, dtype=jnp.int32)
    es = jnp.sum((offsets[None, :] <= (tt * TM)[:, None]).astype(jnp.int32),
                 axis=1) - 1
    ee = jnp.sum((offsets[None, :]
                  <= (tt * TM + (TM - 1))[:, None]).astype(jnp.int32),
                 axis=1) - 1
    ne = ee - es + 1
    start_g = jnp.concatenate(
        [jnp.zeros((1,), jnp.int32), jnp.cumsum(ne, dtype=jnp.int32)])
    total = start_g[-1]
    gg = jnp.arange(G, dtype=jnp.int32)
    tg = jnp.clip(
        jnp.sum((start_g[None, :] <= gg[:, None]).astype(jnp.int32),
                axis=1) - 1,
        0, NT - 1)
    eg = es[tg] + (gg - start_g[tg])
    valid = gg < total
    first = jnp.where(valid, (gg == start_g[tg]), False).astype(jnp.int32)
    tg = jnp.where(valid, tg, NT - 1)
    eg = jnp.where(valid, jnp.clip(eg, 0, E - 1), E - 1)
    valid = valid.astype(jnp.int32)

    pos_e = pos[0::2]
    pos_o = pos[1::2]
    pe3 = pos_e.reshape(_NW, NCH, CH)
    po3 = pos_o.reshape(_NW, NCH, CH)
    pe3e = pos_e.reshape(_NW, NCHE, CHE)
    po3e = pos_o.reshape(_NW, NCHE, CHE)

    Xs = _scatter(xg0, xg1, pe3, po3)
    Xout = _ffn(tg, eg, first, valid, offsets, Xs, W1, W2)
    y = _combine(Xout, pe3e, po3e)
    return y.reshape(B, S, D)


# speedup vs baseline: 1.3360x; 1.0314x over previous
import functools

import jax
import jax.numpy as jnp
from jax import lax
from jax.experimental import pallas as pl
from jax.experimental.pallas import tpu as pltpu
from jax.experimental.pallas import tpu_sc as plsc

B = 4
S = 2048
D = 768
H = 768
E = 64
K = 2
N = B * S
NK = N * K

TB = 512
NA = N // TB


def _router_body(x_ref, wg_ref, ii_ref, pp_ref):
    x = x_ref[:]
    logits = jnp.dot(x, wg_ref[:], preferred_element_type=jnp.float32)
    m = jnp.max(logits, axis=1, keepdims=True)
    ex = jnp.exp(logits - m)
    probs = ex / jnp.sum(ex, axis=1, keepdims=True)
    lane = lax.broadcasted_iota(jnp.int32, (TB, E), 1)
    p0 = jnp.max(probs, axis=1, keepdims=True)
    i0 = jnp.min(jnp.where(probs == p0, lane, E), axis=1, keepdims=True)
    probs2 = jnp.where(lane == i0, -jnp.inf, probs)
    p1 = jnp.max(probs2, axis=1, keepdims=True)
    i1 = jnp.min(jnp.where(probs2 == p1, lane, E), axis=1, keepdims=True)
    ii_ref[:] = jnp.concatenate([i0, i1], axis=1)
    pp_ref[:] = jnp.concatenate([p0, p1], axis=1)


def _router(xf, Wg):
    return pl.pallas_call(
        _router_body,
        grid=(NA,),
        in_specs=[
            pl.BlockSpec((TB, D), lambda i: (i, 0)),
            pl.BlockSpec((D, E), lambda i: (0, 0)),
        ],
        out_specs=[
            pl.BlockSpec((TB, K), lambda i: (i, 0)),
            pl.BlockSpec((TB, K), lambda i: (i, 0)),
        ],
        out_shape=[
            jax.ShapeDtypeStruct((N, K), jnp.int32),
            jax.ShapeDtypeStruct((N, K), jnp.float32),
        ],
    )(xf, Wg)


GC = 256
GR = NK // GC


def _sortpos_body(ids_ref, pos_ref, off_ref):
    ids2d = ids_ref[:]
    c0 = lax.broadcasted_iota(jnp.int32, (GC, GC), 0)
    c1 = lax.broadcasted_iota(jnp.int32, (GC, GC), 1)
    su = (c0 < c1).astype(jnp.bfloat16)
    ones = jnp.ones((GC, GC), jnp.bfloat16)
    g0 = lax.broadcasted_iota(jnp.int32, (GR, GR), 0)
    g1 = lax.broadcasted_iota(jnp.int32, (GR, GR), 1)
    slg = (g1 < g0).astype(jnp.bfloat16)
    rowid = lax.broadcasted_iota(jnp.int32, (GR, GC), 0)

    def step(e, carry):
        pos_acc, off_mat, off_sc = carry
        maskb = (ids2d == e).astype(jnp.bfloat16)
        mask = maskb.astype(jnp.float32)
        rowcnt = jnp.dot(maskb, ones,
                         preferred_element_type=jnp.float32)
        pre = jnp.dot(slg, rowcnt.astype(jnp.bfloat16),
                      preferred_element_type=jnp.float32)
        rank = jnp.dot(maskb, su,
                       preferred_element_type=jnp.float32)
        pos_acc = pos_acc + mask * (off_sc + pre + rank)
        off_mat = off_mat + (rowid == e).astype(jnp.float32) * off_sc
        off_sc = off_sc + jnp.sum(mask)
        return pos_acc, off_mat, off_sc

    z = jnp.zeros((GR, GC), jnp.float32)
    pos_acc, off_mat, _ = lax.fori_loop(0, E, step, (z, z, 0.0))
    pos_ref[:] = pos_acc.astype(jnp.int32)
    off_ref[:] = off_mat.astype(jnp.int32)


def _sortpos(ids2):
    return pl.pallas_call(
        _sortpos_body,
        out_shape=[
            jax.ShapeDtypeStruct((GR, GC), jnp.int32),
            jax.ShapeDtypeStruct((GR, GC), jnp.int32),
        ],
    )(ids2)


TM = 256
NT = NK // TM
G = NT + E - 1


def _ffn_body(tid_ref, eid_ref, first_ref, valid_ref, off_ref,
              x_ref, w1_ref, w2_ref, out_ref):
    g = pl.program_id(0)

    @pl.when(valid_ref[g] == 1)
    def _():
        e = eid_ref[g]
        t = tid_ref[g]
        lo = off_ref[e]
        hi = off_ref[e + 1]
        rows = t * TM + lax.broadcasted_iota(jnp.int32, (TM, 1), 0)
        msk = (rows >= lo) & (rows < hi)
        x = jnp.where(msk, x_ref[:], 0.0)
        h = jnp.maximum(
            jnp.dot(x, w1_ref[0], preferred_element_type=jnp.float32), 0.0)
        part = jnp.dot(h, w2_ref[0], preferred_element_type=jnp.float32)

        @pl.when(first_ref[g] == 1)
        def _():
            out_ref[:] = part

        @pl.when(first_ref[g] == 0)
        def _():
            out_ref[:] = out_ref[:] + part


def _ffn(tid, eid, first, valid, offsets, Xs, W1, W2):
    grid_spec = pltpu.PrefetchScalarGridSpec(
        num_scalar_prefetch=5,
        grid=(G,),
        in_specs=[
            pl.BlockSpec((TM, D), lambda g, t, e, f, v, o: (t[g], 0)),
            pl.BlockSpec((1, D, H), lambda g, t, e, f, v, o: (e[g], 0, 0)),
            pl.BlockSpec((1, H, D), lambda g, t, e, f, v, o: (e[g], 0, 0)),
        ],
        out_specs=pl.BlockSpec((TM, D), lambda g, t, e, f, v, o: (t[g], 0)),
    )
    return pl.pallas_call(
        _ffn_body,
        grid_spec=grid_spec,
        out_shape=jax.ShapeDtypeStruct((NK, D), jnp.float32),
    )(tid, eid, first, valid, offsets, Xs, W1, W2)


_NC, _NS = 2, 16
_NW = _NC * _NS
CH = 64
NCH = (N // _NW) // CH
CHE = 32
NCHE = (N // _NW) // CHE


def _sc_mesh():
    return plsc.VectorSubcoreMesh(core_axis_name="c", subcore_axis_name="s",
                                  num_cores=_NC, num_subcores=_NS)


def _scatter(xf, pe3, po3):
    @functools.partial(
        pl.kernel,
        out_type=jax.ShapeDtypeStruct((NK, D), jnp.float32),
        mesh=_sc_mesh(),
        scratch_types=[
            pltpu.VMEM((CH, D), jnp.float32),
            pltpu.VMEM((CH, D), jnp.float32),
            pltpu.VMEM((NCH, CH), jnp.int32),
            pltpu.VMEM((NCH, CH), jnp.int32),
            pltpu.SemaphoreType.DMA,
            pltpu.SemaphoreType.DMA,
            pltpu.SemaphoreType.DMA,
            pltpu.SemaphoreType.DMA,
            pltpu.SemaphoreType.DMA,
            pltpu.SemaphoreType.DMA,
        ],
    )
    def k(xf_hbm, pe_hbm, po_hbm, out_hbm,
          rb0, rb1, idxe, idxo, ls0, ls1, sa0, sa1, sb0, sb1):
        w = lax.axis_index("s") * _NC + lax.axis_index("c")
        base = w * (N // _NW)
        pltpu.sync_copy(pe_hbm.at[w], idxe)
        pltpu.sync_copy(po_hbm.at[w], idxo)
        bufs, lsems = (rb0, rb1), (ls0, ls1)
        sas, sbs = (sa0, sa1), (sb0, sb1)

        def start_load(j):
            return pltpu.async_copy(
                xf_hbm.at[pl.ds(base + j * CH, CH)], bufs[j % 2],
                lsems[j % 2])

        loads = {0: start_load(0)}
        sca, scb = {}, {}
        for j in range(NCH):
            if j + 1 < NCH:
                if j - 1 >= 0:
                    sca[j - 1].wait()
                    scb[j - 1].wait()
                loads[j + 1] = start_load(j + 1)
            loads[j].wait()
            sca[j] = pltpu.async_copy(
                bufs[j % 2], out_hbm.at[idxe.at[j]], sas[j % 2])
            scb[j] = pltpu.async_copy(
                bufs[j % 2], out_hbm.at[idxo.at[j]], sbs[j % 2])
        for j in (NCH - 2, NCH - 1):
            sca[j].wait()
            scb[j].wait()

    return k(xf, pe3, po3)


def _combine(xout, pe3, po3, p03, p13):
    @functools.partial(
        pl.kernel,
        out_type=jax.ShapeDtypeStruct((N, D), jnp.float32),
        mesh=_sc_mesh(),
        compiler_params=pltpu.CompilerParams(needs_layout_passes=False),
        scratch_types=[
            pltpu.VMEM((CHE, D), jnp.float32),
            pltpu.VMEM((CHE, D), jnp.float32),
            pltpu.VMEM((CHE, D), jnp.float32),
            pltpu.VMEM((CHE, D), jnp.float32),
            pltpu.VMEM((NCHE, CHE), jnp.int32),
            pltpu.VMEM((NCHE, CHE), jnp.int32),
            pltpu.VMEM((NCHE, CHE), jnp.float32),
            pltpu.VMEM((NCHE, CHE), jnp.float32),
            pltpu.SemaphoreType.DMA,
            pltpu.SemaphoreType.DMA,
            pltpu.SemaphoreType.DMA,
            pltpu.SemaphoreType.DMA,
            pltpu.SemaphoreType.DMA,
            pltpu.SemaphoreType.DMA,
        ],
    )
    def k(xout_hbm, pe_hbm, po_hbm, p0_hbm, p1_hbm, y_hbm,
          ge0, ge1, go0, go1, idxe, idxo, pb0, pb1,
          gse0, gse1, gso0, gso1, sts0, sts1):
        w = lax.axis_index("s") * _NC + lax.axis_index("c")
        base = w * (N // _NW)
        pltpu.sync_copy(pe_hbm.at[w], idxe)
        pltpu.sync_copy(po_hbm.at[w], idxo)
        pltpu.sync_copy(p0_hbm.at[w], pb0)
        pltpu.sync_copy(p1_hbm.at[w], pb1)
        ge, go = (ge0, ge1), (go0, go1)
        gse, gso, sts = (gse0, gse1), (gso0, gso1), (sts0, sts1)
        lanes = lax.broadcasted_iota(jnp.int32, (16,), 0)

        def start_gather(j):
            p = j % 2
            return (pltpu.async_copy(xout_hbm.at[idxe.at[j]], ge[p], gse[p]),
                    pltpu.async_copy(xout_hbm.at[idxo.at[j]], go[p], gso[p]))

        gaths = {0: start_gather(0)}
        stores = {}
        for j in range(NCHE):
            p = j % 2
            if j + 1 < NCHE:
                if j - 1 >= 0:
                    stores[j - 1].wait()
                gaths[j + 1] = start_gather(j + 1)
            gaths[j][0].wait()
            gaths[j][1].wait()

            def rowmix(r, carry):
                g16 = 16 * (r // 16)
                lmask = lanes == (r % 16)
                s0 = jnp.sum(jnp.where(lmask, pb0[j, pl.ds(g16, 16)], 0.0))
                s1 = jnp.sum(jnp.where(lmask, pb1[j, pl.ds(g16, 16)], 0.0))
                for c in range(D // 16):
                    sl = pl.ds(c * 16, 16)
                    ge[p][r, sl] = ge[p][r, sl] * s0 + go[p][r, sl] * s1
                return carry

            lax.fori_loop(0, CHE, rowmix, 0)
            stores[j] = pltpu.async_copy(
                ge[p], y_hbm.at[pl.ds(base + j * CHE, CHE)], sts[p])
        stores[NCHE - 2].wait()
        stores[NCHE - 1].wait()

    return k(xout, pe3, po3, p03, p13)


def kernel(inputBatch, Wg, W1, W2):
    xf = inputBatch.reshape(-1, D)
    ii, pp = _router(xf, Wg)

    ids2 = ii.reshape(GR, GC)
    pos2, off2 = _sortpos(ids2)
    pos = pos2.reshape(-1)

    offsets = jnp.concatenate(
        [off2[:, 0], jnp.full((1,), NK, jnp.int32)])

    tt = jnp.arange(NT, dtype=jnp.int32)
    es = jnp.sum((offsets[None, :] <= (tt * TM)[:, None]).astype(jnp.int32),
                 axis=1) - 1
    ee = jnp.sum((offsets[None, :]
                  <= (tt * TM + (TM - 1))[:, None]).astype(jnp.int32),
                 axis=1) - 1
    ne = ee - es + 1
    start_g = jnp.concatenate(
        [jnp.zeros((1,), jnp.int32), jnp.cumsum(ne, dtype=jnp.int32)])
    total = start_g[-1]
    gg = jnp.arange(G, dtype=jnp.int32)
    tg = jnp.clip(
        jnp.sum((start_g[None, :] <= gg[:, None]).astype(jnp.int32),
                axis=1) - 1,
        0, NT - 1)
    eg = es[tg] + (gg - start_g[tg])
    valid = gg < total
    first = jnp.where(valid, (gg == start_g[tg]), False).astype(jnp.int32)
    tg = jnp.where(valid, tg, NT - 1)
    eg = jnp.where(valid, jnp.clip(eg, 0, E - 1), E - 1)
    valid = valid.astype(jnp.int32)

    pos_e = pos[0::2]
    pos_o = pos[1::2]
    pe3 = pos_e.reshape(_NW, NCH, CH)
    po3 = pos_o.reshape(_NW, NCH, CH)
    pe3e = pos_e.reshape(_NW, NCHE, CHE)
    po3e = pos_o.reshape(_NW, NCHE, CHE)
    p03 = pp[:, 0].reshape(_NW, NCHE, CHE)
    p13 = pp[:, 1].reshape(_NW, NCHE, CHE)

    Xs = _scatter(xf, pe3, po3)
    Xout = _ffn(tg, eg, first, valid, offsets, Xs, W1, W2)
    y = _combine(Xout, pe3e, po3e, p03, p13)
    return y.reshape(B, S, D)


# speedup vs baseline: 1.3561x; 1.0150x over previous
import functools

import jax
import jax.numpy as jnp
from jax import lax
from jax.experimental import pallas as pl
from jax.experimental.pallas import tpu as pltpu
from jax.experimental.pallas import tpu_sc as plsc

B = 4
S = 2048
D = 768
H = 768
E = 64
K = 2
N = B * S
NK = N * K

TB = 512
NA = N // TB


def _router_body(x_ref, wg_ref, ii_ref, pp_ref):
    x = x_ref[:]
    logits = jnp.dot(x, wg_ref[:], preferred_element_type=jnp.float32)
    m = jnp.max(logits, axis=1, keepdims=True)
    ex = jnp.exp(logits - m)
    probs = ex / jnp.sum(ex, axis=1, keepdims=True)
    lane = lax.broadcasted_iota(jnp.int32, (TB, E), 1)
    p0 = jnp.max(probs, axis=1, keepdims=True)
    i0 = jnp.min(jnp.where(probs == p0, lane, E), axis=1, keepdims=True)
    probs2 = jnp.where(lane == i0, -jnp.inf, probs)
    p1 = jnp.max(probs2, axis=1, keepdims=True)
    i1 = jnp.min(jnp.where(probs2 == p1, lane, E), axis=1, keepdims=True)
    ii_ref[:] = jnp.concatenate([i0, i1], axis=1)
    pp_ref[:] = jnp.concatenate([p0, p1], axis=1)


def _router(xf, Wg):
    return pl.pallas_call(
        _router_body,
        grid=(NA,),
        in_specs=[
            pl.BlockSpec((TB, D), lambda i: (i, 0)),
            pl.BlockSpec((D, E), lambda i: (0, 0)),
        ],
        out_specs=[
            pl.BlockSpec((TB, K), lambda i: (i, 0)),
            pl.BlockSpec((TB, K), lambda i: (i, 0)),
        ],
        out_shape=[
            jax.ShapeDtypeStruct((N, K), jnp.int32),
            jax.ShapeDtypeStruct((N, K), jnp.float32),
        ],
    )(xf, Wg)


GC = 256
GR = NK // GC


def _sortpos_body(ids_ref, pos_ref, off_ref):
    ids2d = ids_ref[:]
    c0 = lax.broadcasted_iota(jnp.int32, (GC, GC), 0)
    c1 = lax.broadcasted_iota(jnp.int32, (GC, GC), 1)
    su = (c0 < c1).astype(jnp.bfloat16)
    ones = jnp.ones((GC, GC), jnp.bfloat16)
    g0 = lax.broadcasted_iota(jnp.int32, (GR, GR), 0)
    g1 = lax.broadcasted_iota(jnp.int32, (GR, GR), 1)
    slg = (g1 < g0).astype(jnp.bfloat16)
    rowid = lax.broadcasted_iota(jnp.int32, (GR, GC), 0)

    def step(i, carry):
        pos_acc, off_mat, off_sc = carry
        for d in range(2):
            e = 2 * i + d
            maskb = (ids2d == e).astype(jnp.bfloat16)
            mask = maskb.astype(jnp.float32)
            rowcnt = jnp.dot(maskb, ones,
                             preferred_element_type=jnp.float32)
            pre = jnp.dot(slg, rowcnt.astype(jnp.bfloat16),
                          preferred_element_type=jnp.float32)
            rank = jnp.dot(maskb, su,
                           preferred_element_type=jnp.float32)
            pos_acc = pos_acc + mask * (off_sc + pre + rank)
            off_mat = off_mat + (rowid == e).astype(jnp.float32) * off_sc
            off_sc = off_sc + jnp.sum(mask)
        return pos_acc, off_mat, off_sc

    z = jnp.zeros((GR, GC), jnp.float32)
    pos_acc, off_mat, _ = lax.fori_loop(0, E // 2, step, (z, z, 0.0))
    pos_ref[:] = pos_acc.astype(jnp.int32)
    off_ref[:] = off_mat.astype(jnp.int32)


def _sortpos(ids2):
    return pl.pallas_call(
        _sortpos_body,
        out_shape=[
            jax.ShapeDtypeStruct((GR, GC), jnp.int32),
            jax.ShapeDtypeStruct((GR, GC), jnp.int32),
        ],
    )(ids2)


TM = 256
NT = NK // TM
G = NT + E - 1


def _ffn_body(tid_ref, eid_ref, first_ref, valid_ref, off_ref,
              x_ref, w1_ref, w2_ref, out_ref):
    g = pl.program_id(0)

    @pl.when(valid_ref[g] == 1)
    def _():
        e = eid_ref[g]
        t = tid_ref[g]
        lo = off_ref[e]
        hi = off_ref[e + 1]
        rows = t * TM + lax.broadcasted_iota(jnp.int32, (TM, 1), 0)
        msk = (rows >= lo) & (rows < hi)
        x = jnp.where(msk, x_ref[:], 0.0)
        h = jnp.maximum(
            jnp.dot(x, w1_ref[0], preferred_element_type=jnp.float32), 0.0)
        part = jnp.dot(h, w2_ref[0], preferred_element_type=jnp.float32)

        @pl.when(first_ref[g] == 1)
        def _():
            out_ref[:] = part

        @pl.when(first_ref[g] == 0)
        def _():
            out_ref[:] = out_ref[:] + part


def _ffn(tid, eid, first, valid, offsets, Xs, W1, W2):
    grid_spec = pltpu.PrefetchScalarGridSpec(
        num_scalar_prefetch=5,
        grid=(G,),
        in_specs=[
            pl.BlockSpec((TM, D), lambda g, t, e, f, v, o: (t[g], 0)),
            pl.BlockSpec((1, D, H), lambda g, t, e, f, v, o: (e[g], 0, 0)),
            pl.BlockSpec((1, H, D), lambda g, t, e, f, v, o: (e[g], 0, 0)),
        ],
        out_specs=pl.BlockSpec((TM, D), lambda g, t, e, f, v, o: (t[g], 0)),
    )
    return pl.pallas_call(
        _ffn_body,
        grid_spec=grid_spec,
        out_shape=jax.ShapeDtypeStruct((NK, D), jnp.float32),
    )(tid, eid, first, valid, offsets, Xs, W1, W2)


_NC, _NS = 2, 16
_NW = _NC * _NS
CH = 64
NCH = (N // _NW) // CH
CHE = 32
NCHE = (N // _NW) // CHE


def _sc_mesh():
    return plsc.VectorSubcoreMesh(core_axis_name="c", subcore_axis_name="s",
                                  num_cores=_NC, num_subcores=_NS)


def _scatter(xf, pe3, po3):
    @functools.partial(
        pl.kernel,
        out_type=jax.ShapeDtypeStruct((NK, D), jnp.float32),
        mesh=_sc_mesh(),
        scratch_types=[
            pltpu.VMEM((CH, D), jnp.float32),
            pltpu.VMEM((CH, D), jnp.float32),
            pltpu.VMEM((NCH, CH), jnp.int32),
            pltpu.VMEM((NCH, CH), jnp.int32),
            pltpu.SemaphoreType.DMA,
            pltpu.SemaphoreType.DMA,
            pltpu.SemaphoreType.DMA,
            pltpu.SemaphoreType.DMA,
            pltpu.SemaphoreType.DMA,
            pltpu.SemaphoreType.DMA,
        ],
    )
    def k(xf_hbm, pe_hbm, po_hbm, out_hbm,
          rb0, rb1, idxe, idxo, ls0, ls1, sa0, sa1, sb0, sb1):
        w = lax.axis_index("s") * _NC + lax.axis_index("c")
        base = w * (N // _NW)
        pltpu.sync_copy(pe_hbm.at[w], idxe)
        pltpu.sync_copy(po_hbm.at[w], idxo)
        bufs, lsems = (rb0, rb1), (ls0, ls1)
        sas, sbs = (sa0, sa1), (sb0, sb1)

        def start_load(j):
            return pltpu.async_copy(
                xf_hbm.at[pl.ds(base + j * CH, CH)], bufs[j % 2],
                lsems[j % 2])

        loads = {0: start_load(0)}
        sca, scb = {}, {}
        for j in range(NCH):
            if j + 1 < NCH:
                if j - 1 >= 0:
                    sca[j - 1].wait()
                    scb[j - 1].wait()
                loads[j + 1] = start_load(j + 1)
            loads[j].wait()
            sca[j] = pltpu.async_copy(
                bufs[j % 2], out_hbm.at[idxe.at[j]], sas[j % 2])
            scb[j] = pltpu.async_copy(
                bufs[j % 2], out_hbm.at[idxo.at[j]], sbs[j % 2])
        for j in (NCH - 2, NCH - 1):
            sca[j].wait()
            scb[j].wait()

    return k(xf, pe3, po3)


def _combine(xout, pe3, po3, p03, p13):
    @functools.partial(
        pl.kernel,
        out_type=jax.ShapeDtypeStruct((N, D), jnp.float32),
        mesh=_sc_mesh(),
        compiler_params=pltpu.CompilerParams(needs_layout_passes=False),
        scratch_types=[
            pltpu.VMEM((CHE, D), jnp.float32),
            pltpu.VMEM((CHE, D), jnp.float32),
            pltpu.VMEM((CHE, D), jnp.float32),
            pltpu.VMEM((CHE, D), jnp.float32),
            pltpu.VMEM((NCHE, CHE), jnp.int32),
            pltpu.VMEM((NCHE, CHE), jnp.int32),
            pltpu.VMEM((NCHE, CHE), jnp.float32),
            pltpu.VMEM((NCHE, CHE), jnp.float32),
            pltpu.SemaphoreType.DMA,
            pltpu.SemaphoreType.DMA,
            pltpu.SemaphoreType.DMA,
            pltpu.SemaphoreType.DMA,
            pltpu.SemaphoreType.DMA,
            pltpu.SemaphoreType.DMA,
        ],
    )
    def k(xout_hbm, pe_hbm, po_hbm, p0_hbm, p1_hbm, y_hbm,
          ge0, ge1, go0, go1, idxe, idxo, pb0, pb1,
          gse0, gse1, gso0, gso1, sts0, sts1):
        w = lax.axis_index("s") * _NC + lax.axis_index("c")
        base = w * (N // _NW)
        pltpu.sync_copy(pe_hbm.at[w], idxe)
        pltpu.sync_copy(po_hbm.at[w], idxo)
        pltpu.sync_copy(p0_hbm.at[w], pb0)
        pltpu.sync_copy(p1_hbm.at[w], pb1)
        ge, go = (ge0, ge1), (go0, go1)
        gse, gso, sts = (gse0, gse1), (gso0, gso1), (sts0, sts1)
        lanes = lax.broadcasted_iota(jnp.int32, (16,), 0)

        def start_gather(j):
            p = j % 2
            return (pltpu.async_copy(xout_hbm.at[idxe.at[j]], ge[p], gse[p]),
                    pltpu.async_copy(xout_hbm.at[idxo.at[j]], go[p], gso[p]))

        gaths = {0: start_gather(0)}
        stores = {}
        for j in range(NCHE):
            p = j % 2
            if j + 1 < NCHE:
                if j - 1 >= 0:
                    stores[j - 1].wait()
                gaths[j + 1] = start_gather(j + 1)
            gaths[j][0].wait()
            gaths[j][1].wait()

            def rowmix(r, carry):
                g16 = 16 * (r // 16)
                lmask = lanes == (r % 16)
                s0 = jnp.sum(jnp.where(lmask, pb0[j, pl.ds(g16, 16)], 0.0))
                s1 = jnp.sum(jnp.where(lmask, pb1[j, pl.ds(g16, 16)], 0.0))
                for c in range(D // 16):
                    sl = pl.ds(c * 16, 16)
                    ge[p][r, sl] = ge[p][r, sl] * s0 + go[p][r, sl] * s1
                return carry

            lax.fori_loop(0, CHE, rowmix, 0)
            stores[j] = pltpu.async_copy(
                ge[p], y_hbm.at[pl.ds(base + j * CHE, CHE)], sts[p])
        stores[NCHE - 2].wait()
        stores[NCHE - 1].wait()

    return k(xout, pe3, po3, p03, p13)


def kernel(inputBatch, Wg, W1, W2):
    xf = inputBatch.reshape(-1, D)
    ii, pp = _router(xf, Wg)

    ids2 = ii.reshape(GR, GC)
    pos2, off2 = _sortpos(ids2)
    pos = pos2.reshape(-1)

    offsets = jnp.concatenate(
        [off2[:, 0], jnp.full((1,), NK, jnp.int32)])

    tt = jnp.arange(NT, dtype=jnp.int32)
    es = jnp.sum((offsets[None, :] <= (tt * TM)[:, None]).astype(jnp.int32),
                 axis=1) - 1
    ee = jnp.sum((offsets[None, :]
                  <= (tt * TM + (TM - 1))[:, None]).astype(jnp.int32),
                 axis=1) - 1
    ne = ee - es + 1
    start_g = jnp.concatenate(
        [jnp.zeros((1,), jnp.int32), jnp.cumsum(ne, dtype=jnp.int32)])
    total = start_g[-1]
    gg = jnp.arange(G, dtype=jnp.int32)
    tg = jnp.clip(
        jnp.sum((start_g[None, :] <= gg[:, None]).astype(jnp.int32),
                axis=1) - 1,
        0, NT - 1)
    eg = es[tg] + (gg - start_g[tg])
    valid = gg < total
    first = jnp.where(valid, (gg == start_g[tg]), False).astype(jnp.int32)
    tg = jnp.where(valid, tg, NT - 1)
    eg = jnp.where(valid, jnp.clip(eg, 0, E - 1), E - 1)
    valid = valid.astype(jnp.int32)

    pos_e = pos[0::2]
    pos_o = pos[1::2]
    pe3 = pos_e.reshape(_NW, NCH, CH)
    po3 = pos_o.reshape(_NW, NCH, CH)
    pe3e = pos_e.reshape(_NW, NCHE, CHE)
    po3e = pos_o.reshape(_NW, NCHE, CHE)
    p03 = pp[:, 0].reshape(_NW, NCHE, CHE)
    p13 = pp[:, 1].reshape(_NW, NCHE, CHE)

    Xs = _scatter(xf, pe3, po3)
    Xout = _ffn(tg, eg, first, valid, offsets, Xs, W1, W2)
    y = _combine(Xout, pe3e, po3e, p03, p13)
    return y.reshape(B, S, D)
